# Initial kernel scaffold; baseline (speedup 1.0000x reference)
import jax
import jax.numpy as jnp
from jax.experimental import pallas as pl


def kernel(x, edge_index, temp):
    raise NotImplementedError("write your pallas kernel here")


# speedup vs baseline: 3.8758x; 3.8758x over previous
import jax
import jax.numpy as jnp
from jax import lax
from jax.experimental import pallas as pl
from jax.experimental.pallas import tpu as pltpu
from jax.experimental.pallas import tpu_sc as plsc

N = 10000
E = 160000
D = 256
K = 10
NC = 2
NS = 16
L = 16
HALF = D // NC
NPAD = 10240
RT = NPAD // NS
RC = 64
NRC = RT // RC
EC = 128
NCH = 79
ET = NCH * EC
EPAD = NS * ET

_i32 = jnp.int32
_f32 = jnp.float32


def _sc_body(xh, edges, temph,
             th_out, g_out,
             gbuf, thbuf, accbuf, idx2, dloc, tempv,
             acc_s,
             sem):
    c = lax.axis_index("c")
    s = lax.axis_index("s")
    iota = lax.iota(_i32, L)
    row0 = s * RT
    grow0 = c * NPAD + row0

    pltpu.sync_copy(temph, tempv)

    def _fill_gbuf(val):
        v = jnp.full((L,), val, _f32)

        def _fr(r, carry):
            for j in range(HALF // L):
                gbuf[r, pl.ds(j * L, L)] = v
            return carry
        lax.fori_loop(0, EC, _fr, 0)

    def _zero_acc(_, carry):
        _fill_gbuf(0.0)

        def _z(k5, carry2):
            pltpu.sync_copy(gbuf.at[pl.ds(0, RC)],
                            acc_s.at[pl.ds(row0 + k5 * RC, RC)])
            return carry2
        lax.fori_loop(0, NRC, _z, 0)
        return carry

    _zero_acc(0, 0)
    plsc.subcore_barrier()
    _fill_gbuf(1.0)

    def _deg_scatter(j, carry):
        pltpu.sync_copy(edges.at[c, s, j], idx2)
        pltpu.sync_copy(gbuf, acc_s.at[idx2.at[1]], add=True)
        return carry
    lax.fori_loop(0, NCH, _deg_scatter, 0)
    plsc.subcore_barrier()

    def _d_chunk(k5, carry):
        pltpu.sync_copy(acc_s.at[pl.ds(row0 + k5 * RC, RC)], accbuf)

        def _d_group(g, carry2):
            deg = jnp.zeros((L,), _f32)
            for j in range(L):
                rowvec = accbuf[g * L + j, pl.ds(0, L)]
                deg = jnp.where(iota == j, rowvec, deg)
            bits = lax.bitcast_convert_type(deg, _i32)
            y = lax.bitcast_convert_type(0x5F3759DF - (bits >> 1), _f32)
            for _ in range(4):
                y = y * (1.5 - 0.5 * deg * y * y)
            rows = row0 + k5 * RC + g * L + iota
            ok = jnp.logical_and(deg > 0.0, rows < N)
            dloc[pl.ds(k5 * RC + g * L, L)] = jnp.where(ok, y, 0.0)
            return carry2
        lax.fori_loop(0, RC // L, _d_group, 0)
        return carry
    lax.fori_loop(0, NRC, _d_chunk, 0)

    t0 = tempv[0, pl.ds(0, L)]

    def _init_chunk(k5, carry):
        r0g = grow0 + k5 * RC
        pltpu.sync_copy(xh.at[pl.ds(r0g, RC)], accbuf)

        def _init_group(g, carry2):
            dvec = dloc[pl.ds(k5 * RC + g * L, L)]
            for j in range(L):
                r = g * L + j
                dr = jnp.full((L,), dvec[j], _f32)
                for jc in range(HALF // L):
                    cs = pl.ds(jc * L, L)
                    xv = accbuf[r, cs]
                    gbuf[r, cs] = dr * xv
                    thbuf[r, cs] = t0 * xv
            return carry2
        lax.fori_loop(0, RC // L, _init_group, 0)
        pltpu.sync_copy(gbuf.at[pl.ds(0, RC)], g_out.at[pl.ds(r0g, RC)])
        pltpu.sync_copy(thbuf, th_out.at[pl.ds(r0g, RC)])
        return carry
    lax.fori_loop(0, NRC, _init_chunk, 0)
    plsc.subcore_barrier()

    def _hop(k, carry):
        _zero_acc(0, 0)
        plsc.subcore_barrier()

        def _edges(j, carry2):
            pltpu.sync_copy(edges.at[c, s, j], idx2)
            pltpu.async_copy(g_out.at[idx2.at[0]], gbuf, sem).wait()
            pltpu.sync_copy(gbuf, acc_s.at[idx2.at[1]], add=True)
            return carry2
        lax.fori_loop(0, NCH, _edges, 0)
        plsc.subcore_barrier()

        tk = tempv[k + 1, pl.ds(0, L)]

        def _wb_chunk(k5, carry2):
            r0l = row0 + k5 * RC
            r0g = grow0 + k5 * RC
            pltpu.sync_copy(acc_s.at[pl.ds(r0l, RC)], accbuf)
            pltpu.sync_copy(th_out.at[pl.ds(r0g, RC)], thbuf)

            def _wb_group(g, carry3):
                dvec = dloc[pl.ds(k5 * RC + g * L, L)]
                for j in range(L):
                    r = g * L + j
                    dr = jnp.full((L,), dvec[j], _f32)
                    for jc in range(HALF // L):
                        cs = pl.ds(jc * L, L)
                        hv = dr * accbuf[r, cs]
                        thbuf[r, cs] = thbuf[r, cs] + tk * hv
                        gbuf[r, cs] = dr * hv
                return carry3
            lax.fori_loop(0, RC // L, _wb_group, 0)
            pltpu.sync_copy(thbuf, th_out.at[pl.ds(r0g, RC)])
            pltpu.sync_copy(gbuf.at[pl.ds(0, RC)], g_out.at[pl.ds(r0g, RC)])
            return carry2
        lax.fori_loop(0, NRC, _wb_chunk, 0)
        plsc.subcore_barrier()
        return carry
    lax.fori_loop(0, K, _hop, 0)


@jax.jit
def _run(xh, edges, temph):
    mesh = plsc.VectorSubcoreMesh(core_axis_name="c", subcore_axis_name="s")
    f = pl.kernel(
        _sc_body,
        out_type=(
            jax.ShapeDtypeStruct((NC * NPAD, HALF), _f32),
            jax.ShapeDtypeStruct((NC * NPAD, HALF), _f32),
        ),
        mesh=mesh,
        scratch_types=(
            pltpu.VMEM((EC, HALF), _f32),
            pltpu.VMEM((RC, HALF), _f32),
            pltpu.VMEM((RC, HALF), _f32),
            pltpu.VMEM((2, EC), _i32),
            pltpu.VMEM((RT,), _f32),
            pltpu.VMEM((L, L), _f32),
            pltpu.VMEM_SHARED((NPAD, HALF), _f32),
            pltpu.SemaphoreType.DMA,
        ),
    )
    return f(xh, edges, temph)


def kernel(x, edge_index, temp):
    src = edge_index[0].astype(_i32)
    dst = edge_index[1].astype(_i32)
    pad = jnp.full((EPAD - E,), N, _i32)
    src_p = jnp.concatenate([src, pad])
    dst_p = jnp.concatenate([dst, pad])
    edges = jnp.stack([
        jnp.stack([src_p.reshape(NS, NCH, EC),
                   dst_p.reshape(NS, NCH, EC)], axis=2),
        jnp.stack([(src_p + NPAD).reshape(NS, NCH, EC),
                   dst_p.reshape(NS, NCH, EC)], axis=2),
    ])
    xh = x.reshape(N, NC, HALF).transpose(1, 0, 2)
    xh = jnp.pad(xh, ((0, 0), (0, NPAD - N), (0, 0))).reshape(NC * NPAD, HALF)
    temph = jnp.zeros((L, L), _f32).at[: K + 1].set(temp[:, None])
    th, _ = _run(xh, edges, temph)
    th = th.reshape(NC, NPAD, HALF).transpose(1, 0, 2).reshape(NPAD, D)
    return th[:N]


# speedup vs baseline: 4.5011x; 1.1613x over previous
import jax
import jax.numpy as jnp
from jax import lax
from jax.experimental import pallas as pl
from jax.experimental.pallas import tpu as pltpu
from jax.experimental.pallas import tpu_sc as plsc

N = 10000
E = 160000
D = 256
K = 10
NC = 2
NS = 16
L = 16
HALF = D // NC
NPAD = 10240
RT = NPAD // NS
RC = 64
NRC = RT // RC
EC = 128
ECC = 64
NCH = 158
ET = NCH * ECC
EPAD = NS * ET

_i32 = jnp.int32
_f32 = jnp.float32


def _sc_body(xh, edges, temph,
             th_out, g_out,
             gbuf, thbuf, accbuf, idx2, dloc, tempv,
             acc_s,
             semA, semB):
    c = lax.axis_index("c")
    s = lax.axis_index("s")
    iota = lax.iota(_i32, L)
    row0 = s * RT
    grow0 = c * NPAD + row0

    pltpu.sync_copy(temph, tempv)

    def _fill_gbuf(val):
        v = jnp.full((L,), val, _f32)

        def _fr(r, carry):
            for j in range(HALF // L):
                gbuf[r, pl.ds(j * L, L)] = v
            return carry
        lax.fori_loop(0, EC, _fr, 0)

    def _zero_acc(_, carry):
        _fill_gbuf(0.0)

        def _z(k5, carry2):
            pltpu.sync_copy(gbuf.at[pl.ds(0, RC)],
                            acc_s.at[pl.ds(row0 + k5 * RC, RC)])
            return carry2
        lax.fori_loop(0, NRC, _z, 0)
        return carry

    _zero_acc(0, 0)
    plsc.subcore_barrier()
    _fill_gbuf(1.0)

    def _deg_scatter(j, carry):
        pltpu.sync_copy(edges.at[c, s, j], idx2.at[0])
        pltpu.sync_copy(gbuf.at[pl.ds(0, ECC)], acc_s.at[idx2.at[0, 1]],
                        add=True)
        return carry
    lax.fori_loop(0, NCH, _deg_scatter, 0)
    plsc.subcore_barrier()

    def _d_chunk(k5, carry):
        pltpu.sync_copy(acc_s.at[pl.ds(row0 + k5 * RC, RC)], accbuf)

        def _d_group(g, carry2):
            deg = jnp.zeros((L,), _f32)
            for j in range(L):
                rowvec = accbuf[g * L + j, pl.ds(0, L)]
                deg = jnp.where(iota == j, rowvec, deg)
            bits = lax.bitcast_convert_type(deg, _i32)
            y = lax.bitcast_convert_type(0x5F3759DF - (bits >> 1), _f32)
            for _ in range(4):
                y = y * (1.5 - 0.5 * deg * y * y)
            rows = row0 + k5 * RC + g * L + iota
            ok = jnp.logical_and(deg > 0.0, rows < N)
            dloc[pl.ds(k5 * RC + g * L, L)] = jnp.where(ok, y, 0.0)
            return carry2
        lax.fori_loop(0, RC // L, _d_group, 0)
        return carry
    lax.fori_loop(0, NRC, _d_chunk, 0)

    t0 = tempv[0, pl.ds(0, L)]

    def _init_chunk(k5, carry):
        r0g = grow0 + k5 * RC
        pltpu.sync_copy(xh.at[pl.ds(r0g, RC)], accbuf)

        def _init_group(g, carry2):
            dvec = dloc[pl.ds(k5 * RC + g * L, L)]
            for j in range(L):
                r = g * L + j
                dr = jnp.full((L,), dvec[j], _f32)
                for jc in range(HALF // L):
                    cs = pl.ds(jc * L, L)
                    xv = accbuf[r, cs]
                    gbuf[r, cs] = dr * xv
                    thbuf[r, cs] = t0 * xv
            return carry2
        lax.fori_loop(0, RC // L, _init_group, 0)
        pltpu.sync_copy(gbuf.at[pl.ds(0, RC)], g_out.at[pl.ds(r0g, RC)])
        pltpu.sync_copy(thbuf, th_out.at[pl.ds(r0g, RC)])
        return carry
    lax.fori_loop(0, NRC, _init_chunk, 0)
    plsc.subcore_barrier()

    def _hop(k, carry):
        _zero_acc(0, 0)
        plsc.subcore_barrier()

        slotA = gbuf.at[pl.ds(0, ECC)]
        slotB = gbuf.at[pl.ds(ECC, ECC)]
        pltpu.sync_copy(edges.at[c, s, 0], idx2.at[0])
        pltpu.async_copy(g_out.at[idx2.at[0, 0]], slotA, semA)

        def _edges(m, carry2):
            pltpu.sync_copy(edges.at[c, s, 2 * m + 1], idx2.at[1])
            pltpu.async_copy(g_out.at[idx2.at[1, 0]], slotB, semB)
            pltpu.make_async_copy(g_out.at[idx2.at[0, 0]], slotA, semA).wait()
            pltpu.sync_copy(slotA, acc_s.at[idx2.at[0, 1]], add=True)

            @pl.when(m < NCH // 2 - 1)
            def _prefetch():
                pltpu.sync_copy(edges.at[c, s, 2 * m + 2], idx2.at[0])
                pltpu.async_copy(g_out.at[idx2.at[0, 0]], slotA, semA)

            pltpu.make_async_copy(g_out.at[idx2.at[1, 0]], slotB, semB).wait()
            pltpu.sync_copy(slotB, acc_s.at[idx2.at[1, 1]], add=True)
            return carry2
        lax.fori_loop(0, NCH // 2, _edges, 0)
        plsc.subcore_barrier()

        tk = tempv[k + 1, pl.ds(0, L)]

        def _wb_chunk(k5, carry2):
            r0l = row0 + k5 * RC
            r0g = grow0 + k5 * RC
            pltpu.sync_copy(acc_s.at[pl.ds(r0l, RC)], accbuf)
            pltpu.sync_copy(th_out.at[pl.ds(r0g, RC)], thbuf)

            def _wb_group(g, carry3):
                dvec = dloc[pl.ds(k5 * RC + g * L, L)]
                for j in range(L):
                    r = g * L + j
                    dr = jnp.full((L,), dvec[j], _f32)
                    for jc in range(HALF // L):
                        cs = pl.ds(jc * L, L)
                        hv = dr * accbuf[r, cs]
                        thbuf[r, cs] = thbuf[r, cs] + tk * hv
                        gbuf[r, cs] = dr * hv
                return carry3
            lax.fori_loop(0, RC // L, _wb_group, 0)
            pltpu.sync_copy(thbuf, th_out.at[pl.ds(r0g, RC)])
            pltpu.sync_copy(gbuf.at[pl.ds(0, RC)], g_out.at[pl.ds(r0g, RC)])
            return carry2
        lax.fori_loop(0, NRC, _wb_chunk, 0)
        plsc.subcore_barrier()
        return carry
    lax.fori_loop(0, K, _hop, 0)


@jax.jit
def _run(xh, edges, temph):
    mesh = plsc.VectorSubcoreMesh(core_axis_name="c", subcore_axis_name="s")
    f = pl.kernel(
        _sc_body,
        out_type=(
            jax.ShapeDtypeStruct((NC * NPAD, HALF), _f32),
            jax.ShapeDtypeStruct((NC * NPAD, HALF), _f32),
        ),
        mesh=mesh,
        scratch_types=(
            pltpu.VMEM((EC, HALF), _f32),
            pltpu.VMEM((RC, HALF), _f32),
            pltpu.VMEM((RC, HALF), _f32),
            pltpu.VMEM((2, 2, ECC), _i32),
            pltpu.VMEM((RT,), _f32),
            pltpu.VMEM((L, L), _f32),
            pltpu.VMEM_SHARED((NPAD, HALF), _f32),
            pltpu.SemaphoreType.DMA,
            pltpu.SemaphoreType.DMA,
        ),
    )
    return f(xh, edges, temph)


def kernel(x, edge_index, temp):
    src = edge_index[0].astype(_i32)
    dst = edge_index[1].astype(_i32)
    pad = jnp.full((EPAD - E,), N, _i32)
    src_p = jnp.concatenate([src, pad])
    dst_p = jnp.concatenate([dst, pad])
    edges = jnp.stack([
        jnp.stack([src_p.reshape(NS, NCH, ECC),
                   dst_p.reshape(NS, NCH, ECC)], axis=2),
        jnp.stack([(src_p + NPAD).reshape(NS, NCH, ECC),
                   dst_p.reshape(NS, NCH, ECC)], axis=2),
    ])
    xh = x.reshape(N, NC, HALF).transpose(1, 0, 2)
    xh = jnp.pad(xh, ((0, 0), (0, NPAD - N), (0, 0))).reshape(NC * NPAD, HALF)
    temph = jnp.zeros((L, L), _f32).at[: K + 1].set(temp[:, None])
    th, _ = _run(xh, edges, temph)
    th = th.reshape(NC, NPAD, HALF).transpose(1, 0, 2).reshape(NPAD, D)
    return th[:N]
